# Initial kernel scaffold; baseline (speedup 1.0000x reference)
import jax
import jax.numpy as jnp
from jax.experimental import pallas as pl


def kernel(x, edge_index, edge_weight, Wq, bq, Wk, bk, Wv, bv):
    raise NotImplementedError("write your pallas kernel here")


# speedup vs baseline: 3.2131x; 3.2131x over previous
import functools

import jax
import jax.numpy as jnp
from jax import lax
from jax.experimental import pallas as pl
from jax.experimental.pallas import tpu as pltpu
from jax.experimental.pallas import tpu_sc as plsc

_NC = 2
_NS = 16
_NW = _NC * _NS
_L = 16
_CH = 80


@functools.cache
def _build(n, e, d):
    f32 = jnp.float32
    ew_per = e // _NW
    n_chunks = ew_per // _CH
    assert ew_per * _NW == e and n_chunks * _CH == ew_per
    assert d % _L == 0 and n % _NS == 0
    rows_per_tile = n // _NS
    inv_scale = 1.0 / (d ** 0.5)
    mesh = plsc.VectorSubcoreMesh(core_axis_name="c", subcore_axis_name="s")
    sc_params = pltpu.CompilerParams(needs_layout_passes=False)

    bn = 2000
    dn = (((1,), (1,)), ((), ()))

    def qkv_body(x_ref, wq_ref, bq_ref, wk_ref, bk_ref, wv_ref, bv_ref,
                 q_ref, k_ref, v_ref):
        xb = x_ref[...]
        q_ref[...] = lax.dot_general(xb, wq_ref[...], dn,
                                     preferred_element_type=f32) + bq_ref[...]
        k_ref[...] = lax.dot_general(xb, wk_ref[...], dn,
                                     preferred_element_type=f32) + bk_ref[...]
        v_ref[...] = lax.dot_general(xb, wv_ref[...], dn,
                                     preferred_element_type=f32) + bv_ref[...]

    mat = pl.BlockSpec((d, d), lambda i: (0, 0))
    vec = pl.BlockSpec((d,), lambda i: (0,))
    rows = pl.BlockSpec((bn, d), lambda i: (i, 0))
    qkv = pl.pallas_call(
        qkv_body,
        grid=(n // bn,),
        in_specs=[rows, mat, vec, mat, vec, mat, vec],
        out_specs=[rows, rows, rows],
        out_shape=[jax.ShapeDtypeStruct((n, d), f32)] * 3,
    )

    @functools.partial(
        pl.kernel, mesh=mesh, compiler_params=sc_params,
        out_type=[jax.ShapeDtypeStruct((e,), f32),
                  jax.ShapeDtypeStruct((_NW * _L,), f32)],
        scratch_types=[
            pltpu.VMEM((_CH,), jnp.int32),
            pltpu.VMEM((_CH,), jnp.int32),
            pltpu.VMEM((_CH, d), f32),
            pltpu.VMEM((_CH, d), f32),
            pltpu.VMEM((_CH,), f32),
            pltpu.VMEM((_CH,), f32),
            pltpu.VMEM((_L,), f32),
            pltpu.SemaphoreType.DMA,
            pltpu.SemaphoreType.DMA,
        ],
    )
    def pass1(row_hbm, col_hbm, ew_hbm, q_hbm, k_hbm,
              z_hbm, mx_hbm,
              ridx, cidx, qg, kg, ewv, zb, mxb, sem_q, sem_k):
        wid = lax.axis_index("s") * _NC + lax.axis_index("c")
        base = wid * ew_per

        def chunk(ci, mx):
            off = base + ci * _CH
            pltpu.sync_copy(row_hbm.at[pl.ds(off, _CH)], ridx)
            pltpu.sync_copy(col_hbm.at[pl.ds(off, _CH)], cidx)
            pltpu.sync_copy(ew_hbm.at[pl.ds(off, _CH)], ewv)
            cq = pltpu.async_copy(q_hbm.at[ridx], qg, sem_q)
            ck = pltpu.async_copy(k_hbm.at[cidx], kg, sem_k)
            cq.wait()
            ck.wait()
            masks = [lax.iota(jnp.int32, _L) == j for j in range(_L)]

            def gbody(g, mx):
                zvec = jnp.zeros((_L,), f32)
                for j in range(_L):
                    e = g * _L + j
                    acc = jnp.zeros((_L,), f32)
                    for c in range(d // _L):
                        acc = acc + (qg[e, pl.ds(c * _L, _L)] *
                                     kg[e, pl.ds(c * _L, _L)])
                    zvec = jnp.where(masks[j], jnp.sum(acc), zvec)
                zg = zvec * (ewv[pl.ds(g * _L, _L)] * inv_scale)
                zb[pl.ds(g * _L, _L)] = zg
                return jnp.maximum(mx, zg)

            mx = lax.fori_loop(0, _CH // _L, gbody, mx)
            pltpu.sync_copy(zb, z_hbm.at[pl.ds(off, _CH)])
            return mx

        mx = lax.fori_loop(0, n_chunks, chunk, jnp.full((_L,), -3e38, f32))
        mxb[...] = jnp.full((_L,), jnp.max(mx), f32)
        pltpu.sync_copy(mxb, mx_hbm.at[pl.ds(wid * _L, _L)])

    rpt = (n // _NS) & ~7
    tail0 = rpt * _NS
    tail = n - tail0
    zrows = 208
    assert rpt % zrows == 0 and tail % 8 == 0

    @functools.partial(
        pl.kernel, mesh=mesh, compiler_params=sc_params,
        out_type=[jax.ShapeDtypeStruct((_NC, n, d), f32),
                  jax.ShapeDtypeStruct((_NW * _L,), f32)],
        scratch_types=[
            pltpu.VMEM((_CH,), jnp.int32),
            pltpu.VMEM((_CH,), jnp.int32),
            pltpu.VMEM((_CH, d), f32),
            pltpu.VMEM((_CH,), f32),
            pltpu.VMEM((_NW * _L,), f32),
            pltpu.VMEM((_L,), f32),
            pltpu.VMEM((zrows, d), f32),
            pltpu.VMEM_SHARED((n, d), f32),
            pltpu.SemaphoreType.DMA,
        ],
    )
    def pass2(row_hbm, col_hbm, z_hbm, mx_hbm, v_hbm,
              part_hbm, se_hbm,
              ridx, cidx, vg, zbuf, mxv, stage, zrb, accum, sem_v):
        cid = lax.axis_index("c")
        sid = lax.axis_index("s")
        wid = sid * _NC + cid
        base = wid * ew_per
        row0 = sid * rpt

        def zfill(i, _):
            for j in range(d // _L):
                zrb[i, pl.ds(j * _L, _L)] = jnp.zeros((_L,), f32)
            return 0

        lax.fori_loop(0, zrows, zfill, 0)
        for r in range(rpt // zrows):
            pltpu.sync_copy(zrb, accum.at[pl.ds(row0 + r * zrows, zrows)])

        @pl.when(sid == _NS - 1)
        def _():
            pltpu.sync_copy(zrb.at[pl.ds(0, tail)],
                            accum.at[pl.ds(tail0, tail)])

        plsc.subcore_barrier()

        pltpu.sync_copy(mx_hbm, mxv)
        m = jnp.full((_L,), -3e38, f32)
        for i in range(_NW):
            m = jnp.maximum(m, mxv[pl.ds(i * _L, _L)])
        gmax = jnp.max(m)

        def chunk(ci, seacc):
            off = base + ci * _CH
            pltpu.sync_copy(row_hbm.at[pl.ds(off, _CH)], ridx)
            pltpu.sync_copy(col_hbm.at[pl.ds(off, _CH)], cidx)
            pltpu.sync_copy(z_hbm.at[pl.ds(off, _CH)], zbuf)
            cv = pltpu.async_copy(v_hbm.at[cidx], vg, sem_v)
            cv.wait()

            def gbody(g, seacc):
                w = jnp.exp(zbuf[pl.ds(g * _L, _L)] - gmax)
                for j in range(_L):
                    e = g * _L + j
                    s = w[j]
                    for c in range(d // _L):
                        vg[e, pl.ds(c * _L, _L)] = (
                            vg[e, pl.ds(c * _L, _L)] * s)
                return seacc + w

            seacc = lax.fori_loop(0, _CH // _L, gbody, seacc)
            pltpu.sync_copy(vg, accum.at[ridx], add=True)
            return seacc

        seacc = lax.fori_loop(0, n_chunks, chunk, jnp.zeros((_L,), f32))
        stage[...] = jnp.full((_L,), jnp.sum(seacc), f32)
        pltpu.sync_copy(stage, se_hbm.at[pl.ds(wid * _L, _L)])

        plsc.subcore_barrier()
        pltpu.sync_copy(accum.at[pl.ds(row0, rpt)],
                        part_hbm.at[cid, pl.ds(row0, rpt)])

        @pl.when(sid == _NS - 1)
        def _():
            pltpu.sync_copy(accum.at[pl.ds(tail0, tail)],
                            part_hbm.at[cid, pl.ds(tail0, tail)])

    def comb_body(p_ref, se_ref, out_ref):
        s = jnp.sum(se_ref[...]) * (1.0 / _L)
        out_ref[...] = (p_ref[0] + p_ref[1]) * (1.0 / s)

    comb = pl.pallas_call(
        comb_body,
        grid=(n // bn,),
        in_specs=[pl.BlockSpec((_NC, bn, d), lambda i: (0, i, 0)),
                  pl.BlockSpec((_NW * _L,), lambda i: (0,))],
        out_specs=pl.BlockSpec((bn, d), lambda i: (i, 0)),
        out_shape=jax.ShapeDtypeStruct((n, d), f32),
    )

    return qkv, pass1, pass2, comb


def kernel(x, edge_index, edge_weight, Wq, bq, Wk, bk, Wv, bv):
    n, d = x.shape
    e = edge_weight.shape[0]
    qkv, pass1, pass2, comb = _build(n, e, d)
    row = edge_index[0]
    col = edge_index[1]
    q, k, v = qkv(x, Wq, bq, Wk, bk, Wv, bv)
    z, mx = pass1(row, col, edge_weight, q, k)
    part, se = pass2(row, col, z, mx, v)
    return comb(part, se)


# speedup vs baseline: 4.2231x; 1.3143x over previous
import functools

import jax
import jax.numpy as jnp
from jax import lax
from jax.experimental import pallas as pl
from jax.experimental.pallas import tpu as pltpu
from jax.experimental.pallas import tpu_sc as plsc

_NC = 2
_NS = 16
_NW = _NC * _NS
_L = 16
_CH = 80


@functools.cache
def _build(n, e, d):
    f32 = jnp.float32
    ew_per = e // _NW
    n_chunks = ew_per // _CH
    assert ew_per * _NW == e and n_chunks * _CH == ew_per
    assert d % _L == 0 and n % _NS == 0
    assert n_chunks % 2 == 1
    rows_per_tile = n // _NS
    inv_scale = 1.0 / (d ** 0.5)
    mesh = plsc.VectorSubcoreMesh(core_axis_name="c", subcore_axis_name="s")
    sc_params = pltpu.CompilerParams(needs_layout_passes=False)

    bn = 2000
    dn = (((1,), (1,)), ((), ()))

    def qkv_body(x_ref, wq_ref, bq_ref, wk_ref, bk_ref, wv_ref, bv_ref,
                 q_ref, k_ref, v_ref):
        xb = x_ref[...]
        q_ref[...] = lax.dot_general(xb, wq_ref[...], dn,
                                     preferred_element_type=f32) + bq_ref[...]
        k_ref[...] = lax.dot_general(xb, wk_ref[...], dn,
                                     preferred_element_type=f32) + bk_ref[...]
        v_ref[...] = lax.dot_general(xb, wv_ref[...], dn,
                                     preferred_element_type=f32) + bv_ref[...]

    mat = pl.BlockSpec((d, d), lambda i: (0, 0))
    vec = pl.BlockSpec((d,), lambda i: (0,))
    rows = pl.BlockSpec((bn, d), lambda i: (i, 0))
    qkv = pl.pallas_call(
        qkv_body,
        grid=(n // bn,),
        in_specs=[rows, mat, vec, mat, vec, mat, vec],
        out_specs=[rows, rows, rows],
        out_shape=[jax.ShapeDtypeStruct((n, d), f32)] * 3,
    )

    @functools.partial(
        pl.kernel, mesh=mesh, compiler_params=sc_params,
        out_type=[jax.ShapeDtypeStruct((e,), f32),
                  jax.ShapeDtypeStruct((_NW * _L,), f32)],
        scratch_types=[
            pltpu.VMEM((2, _CH), jnp.int32),
            pltpu.VMEM((2, _CH), jnp.int32),
            pltpu.VMEM((2, _CH, d), f32),
            pltpu.VMEM((2, _CH, d), f32),
            pltpu.VMEM((2, _CH), f32),
            pltpu.VMEM((_CH,), f32),
            pltpu.VMEM((_L,), f32),
            pltpu.SemaphoreType.DMA,
            pltpu.SemaphoreType.DMA,
            pltpu.SemaphoreType.DMA,
            pltpu.SemaphoreType.DMA,
        ],
    )
    def pass1(row_hbm, col_hbm, ew_hbm, q_hbm, k_hbm,
              z_hbm, mx_hbm,
              ridx, cidx, qg, kg, ewv, zb, mxb,
              sem_q0, sem_k0, sem_q1, sem_k1):
        wid = lax.axis_index("s") * _NC + lax.axis_index("c")
        base = wid * ew_per
        sems = ((sem_q0, sem_k0), (sem_q1, sem_k1))
        masks = [lax.iota(jnp.int32, _L) == j for j in range(_L)]

        def load(ci, b):
            off = base + ci * _CH
            pltpu.sync_copy(row_hbm.at[pl.ds(off, _CH)], ridx.at[b])
            pltpu.sync_copy(col_hbm.at[pl.ds(off, _CH)], cidx.at[b])
            pltpu.sync_copy(ew_hbm.at[pl.ds(off, _CH)], ewv.at[b])
            pltpu.async_copy(q_hbm.at[ridx.at[b]], qg.at[b], sems[b][0])
            pltpu.async_copy(k_hbm.at[cidx.at[b]], kg.at[b], sems[b][1])

        def wait(b):
            pltpu.make_async_copy(q_hbm.at[ridx.at[b]], qg.at[b],
                                  sems[b][0]).wait()
            pltpu.make_async_copy(k_hbm.at[cidx.at[b]], kg.at[b],
                                  sems[b][1]).wait()

        def compute(ci, b, mx):
            off = base + ci * _CH

            def gbody(g, mx):
                zvec = jnp.zeros((_L,), f32)
                for j in range(_L):
                    ed = g * _L + j
                    acc = jnp.zeros((_L,), f32)
                    for c in range(d // _L):
                        acc = acc + (qg[b, ed, pl.ds(c * _L, _L)] *
                                     kg[b, ed, pl.ds(c * _L, _L)])
                    zvec = jnp.where(masks[j], jnp.sum(acc), zvec)
                zg = zvec * (ewv[b, pl.ds(g * _L, _L)] * inv_scale)
                zb[pl.ds(g * _L, _L)] = zg
                return jnp.maximum(mx, zg)

            mx = lax.fori_loop(0, _CH // _L, gbody, mx)
            pltpu.sync_copy(zb, z_hbm.at[pl.ds(off, _CH)])
            return mx

        load(0, 0)

        def pair(p, mx):
            ci = 2 * p
            load(ci + 1, 1)
            wait(0)
            mx = compute(ci, 0, mx)
            load(ci + 2, 0)
            wait(1)
            mx = compute(ci + 1, 1, mx)
            return mx

        mx = lax.fori_loop(0, (n_chunks - 1) // 2, pair,
                           jnp.full((_L,), -3e38, f32))
        wait(0)
        mx = compute(n_chunks - 1, 0, mx)
        mxb[...] = jnp.full((_L,), jnp.max(mx), f32)
        pltpu.sync_copy(mxb, mx_hbm.at[pl.ds(wid * _L, _L)])

    rpt = (n // _NS) & ~7
    tail0 = rpt * _NS
    tail = n - tail0
    zrows = 208
    assert rpt % zrows == 0 and tail % 8 == 0

    @functools.partial(
        pl.kernel, mesh=mesh, compiler_params=sc_params,
        out_type=[jax.ShapeDtypeStruct((_NC, n, d), f32),
                  jax.ShapeDtypeStruct((_NW * _L,), f32)],
        scratch_types=[
            pltpu.VMEM((2, _CH), jnp.int32),
            pltpu.VMEM((2, _CH), jnp.int32),
            pltpu.VMEM((2, _CH, d), f32),
            pltpu.VMEM((2, _CH), f32),
            pltpu.VMEM((_NW * _L,), f32),
            pltpu.VMEM((_L,), f32),
            pltpu.VMEM((zrows, d), f32),
            pltpu.VMEM_SHARED((n, d), f32),
            pltpu.SemaphoreType.DMA,
            pltpu.SemaphoreType.DMA,
        ],
    )
    def pass2(row_hbm, col_hbm, z_hbm, mx_hbm, v_hbm,
              part_hbm, se_hbm,
              ridx, cidx, vg, zbuf, mxv, stage, zrb, accum,
              sem_v0, sem_v1):
        cid = lax.axis_index("c")
        sid = lax.axis_index("s")
        wid = sid * _NC + cid
        base = wid * ew_per
        row0 = sid * rpt
        sems = (sem_v0, sem_v1)

        def zfill(i, _):
            for j in range(d // _L):
                zrb[i, pl.ds(j * _L, _L)] = jnp.zeros((_L,), f32)
            return 0

        lax.fori_loop(0, zrows, zfill, 0)
        for r in range(rpt // zrows):
            pltpu.sync_copy(zrb, accum.at[pl.ds(row0 + r * zrows, zrows)])

        @pl.when(sid == _NS - 1)
        def _():
            pltpu.sync_copy(zrb.at[pl.ds(0, tail)],
                            accum.at[pl.ds(tail0, tail)])

        plsc.subcore_barrier()

        pltpu.sync_copy(mx_hbm, mxv)
        m = jnp.full((_L,), -3e38, f32)
        for i in range(_NW):
            m = jnp.maximum(m, mxv[pl.ds(i * _L, _L)])
        gmax = jnp.max(m)

        def load(ci, b):
            off = base + ci * _CH
            pltpu.sync_copy(row_hbm.at[pl.ds(off, _CH)], ridx.at[b])
            pltpu.sync_copy(col_hbm.at[pl.ds(off, _CH)], cidx.at[b])
            pltpu.sync_copy(z_hbm.at[pl.ds(off, _CH)], zbuf.at[b])
            pltpu.async_copy(v_hbm.at[cidx.at[b]], vg.at[b], sems[b])

        def wait(b):
            pltpu.make_async_copy(v_hbm.at[cidx.at[b]], vg.at[b],
                                  sems[b]).wait()

        def compute(b, seacc):
            def gbody(g, seacc):
                w = jnp.exp(zbuf[b, pl.ds(g * _L, _L)] - gmax)
                for j in range(_L):
                    ed = g * _L + j
                    s = w[j]
                    for c in range(d // _L):
                        vg[b, ed, pl.ds(c * _L, _L)] = (
                            vg[b, ed, pl.ds(c * _L, _L)] * s)
                return seacc + w

            seacc = lax.fori_loop(0, _CH // _L, gbody, seacc)
            pltpu.sync_copy(vg.at[b], accum.at[ridx.at[b]], add=True)
            return seacc

        load(0, 0)

        def pair(p, seacc):
            ci = 2 * p
            load(ci + 1, 1)
            wait(0)
            seacc = compute(0, seacc)
            load(ci + 2, 0)
            wait(1)
            seacc = compute(1, seacc)
            return seacc

        seacc = lax.fori_loop(0, (n_chunks - 1) // 2, pair,
                              jnp.zeros((_L,), f32))
        wait(0)
        seacc = compute(0, seacc)
        stage[...] = jnp.full((_L,), jnp.sum(seacc), f32)
        pltpu.sync_copy(stage, se_hbm.at[pl.ds(wid * _L, _L)])

        plsc.subcore_barrier()
        pltpu.sync_copy(accum.at[pl.ds(row0, rpt)],
                        part_hbm.at[cid, pl.ds(row0, rpt)])

        @pl.when(sid == _NS - 1)
        def _():
            pltpu.sync_copy(accum.at[pl.ds(tail0, tail)],
                            part_hbm.at[cid, pl.ds(tail0, tail)])

    def comb_body(p_ref, se_ref, out_ref):
        s = jnp.sum(se_ref[...]) * (1.0 / _L)
        out_ref[...] = (p_ref[0] + p_ref[1]) * (1.0 / s)

    comb = pl.pallas_call(
        comb_body,
        grid=(n // bn,),
        in_specs=[pl.BlockSpec((_NC, bn, d), lambda i: (0, i, 0)),
                  pl.BlockSpec((_NW * _L,), lambda i: (0,))],
        out_specs=pl.BlockSpec((bn, d), lambda i: (i, 0)),
        out_shape=jax.ShapeDtypeStruct((n, d), f32),
    )

    return qkv, pass1, pass2, comb


def kernel(x, edge_index, edge_weight, Wq, bq, Wk, bk, Wv, bv):
    n, d = x.shape
    e = edge_weight.shape[0]
    qkv, pass1, pass2, comb = _build(n, e, d)
    row = edge_index[0]
    col = edge_index[1]
    q, k, v = qkv(x, Wq, bq, Wk, bk, Wv, bv)
    z, mx = pass1(row, col, edge_weight, q, k)
    part, se = pass2(row, col, z, mx, v)
    return comb(part, se)
